# baseline (speedup 1.0000x reference)
import dataclasses
import functools

import jax
import jax.numpy as jnp
from jax import lax
from jax.experimental import pallas as pl
from jax.experimental.pallas import tpu as pltpu
from jax.experimental.pallas import tpu_sc as plsc

BATCH = 16384
EMBED_DIM = 64

NC = 2
NS = 16
NW = NC * NS
BPW = BATCH // NW
LANES = 16
VPW = BPW // LANES

_sc_mesh = plsc.VectorSubcoreMesh(core_axis_name="c", subcore_axis_name="s")

_sc_params = pltpu.CompilerParams()
if "needs_layout_passes" in pltpu.CompilerParams.__dataclass_fields__:
    _sc_params = dataclasses.replace(_sc_params, needs_layout_passes=False)


@jax.jit
def _sc_gather(user_table, product_table, uidx, pidx):
    @functools.partial(
        pl.kernel,
        mesh=_sc_mesh,
        compiler_params=_sc_params,
        out_type=(
            jax.ShapeDtypeStruct((BATCH, EMBED_DIM), jnp.float32),
            jax.ShapeDtypeStruct((BATCH, EMBED_DIM), jnp.float32),
        ),
        scratch_types=[
            pltpu.VMEM((VPW, LANES), jnp.int32),
            pltpu.VMEM((VPW, LANES), jnp.int32),
            pltpu.SemaphoreType.DMA,
            pltpu.SemaphoreType.DMA,
        ],
    )
    def k(ut_hbm, pt_hbm, ui_hbm, pi_hbm, ue_hbm, pe_hbm,
          ui_v, pi_v, usem, psem):
        wid = lax.axis_index("s") * NC + lax.axis_index("c")
        base = wid * BPW
        pltpu.sync_copy(ui_hbm.at[pl.ds(wid * VPW, VPW)], ui_v)
        pltpu.sync_copy(pi_hbm.at[pl.ds(wid * VPW, VPW)], pi_v)
        lane = lax.iota(jnp.int32, LANES)

        @pl.loop(0, VPW)
        def _(c):
            uvec = ui_v[c, :]
            pvec = pi_v[c, :]
            row0 = base + c * LANES
            for l in range(LANES):
                r = jnp.sum(jnp.where(lane == l, uvec, 0))
                pltpu.make_async_copy(
                    ut_hbm.at[pl.ds(r, 1)], ue_hbm.at[pl.ds(row0 + l, 1)],
                    usem).start()
                q = jnp.sum(jnp.where(lane == l, pvec, 0))
                pltpu.make_async_copy(
                    pt_hbm.at[pl.ds(q, 1)], pe_hbm.at[pl.ds(row0 + l, 1)],
                    psem).start()

        pltpu.make_async_copy(
            ut_hbm.at[pl.ds(0, BPW)], ue_hbm.at[pl.ds(base, BPW)], usem).wait()
        pltpu.make_async_copy(
            pt_hbm.at[pl.ds(0, BPW)], pe_hbm.at[pl.ds(base, BPW)], psem).wait()

    return k(user_table, product_table, uidx, pidx)


def _mlp_body(ue, pe, uf, bd, w1, b1, w2, b2, w3a, w3b, w3c, b3, w4, b4, out):
    m = ue[...] * pe[...]
    ufeat = jnp.maximum(
        jnp.dot(uf[...], w1[...], preferred_element_type=jnp.float32) + b1[...], 0.0)
    bfeat = jnp.maximum(
        jnp.dot(bd[...], w2[...], preferred_element_type=jnp.float32) + b2[...], 0.0)
    h = (jnp.dot(m, w3a[...], preferred_element_type=jnp.float32)
         + jnp.dot(ufeat, w3b[...], preferred_element_type=jnp.float32)
         + jnp.dot(bfeat, w3c[...], preferred_element_type=jnp.float32)
         + b3[...])
    h = jnp.maximum(h, 0.0)
    logit = jnp.dot(h, w4[...], preferred_element_type=jnp.float32) + b4[...]
    out[...] = jax.nn.sigmoid(logit)


_TC_BLOCK = 2048


@jax.jit
def _tc_mlp(ue, pe, uf, bd, w1, b1, w2, b2, w3a, w3b, w3c, b3, w4, b4):
    def row_block(width):
        return pl.BlockSpec((_TC_BLOCK, width), lambda i: (i, 0))

    def whole(a):
        return pl.BlockSpec(a.shape, lambda i: (0, 0))

    return pl.pallas_call(
        _mlp_body,
        grid=(BATCH // _TC_BLOCK,),
        in_specs=[row_block(EMBED_DIM), row_block(EMBED_DIM), row_block(11),
                  row_block(3),
                  whole(w1), whole(b1), whole(w2), whole(b2),
                  whole(w3a), whole(w3b), whole(w3c), whole(b3),
                  whole(w4), whole(b4)],
        out_specs=row_block(1),
        out_shape=jax.ShapeDtypeStruct((BATCH, 1), jnp.float32),
    )(ue, pe, uf, bd, w1, b1, w2, b2, w3a, w3b, w3c, b3, w4, b4)


def kernel(user_ids, product_ids, user_features, behavior_data,
           user_table, product_table, W1, b1, W2, b2, W3, b3, W4, b4):
    ui = user_ids.reshape(BATCH // LANES, LANES)
    pi = product_ids.reshape(BATCH // LANES, LANES)
    ue, pe = _sc_gather(user_table, product_table, ui, pi)
    return _tc_mlp(
        ue, pe, user_features, behavior_data,
        W1.T, b1.reshape(1, 32), W2.T, b2.reshape(1, 32),
        W3[:, :EMBED_DIM].T, W3[:, EMBED_DIM:EMBED_DIM + 32].T,
        W3[:, EMBED_DIM + 32:].T, b3.reshape(1, 32),
        W4.T, b4.reshape(1, 1))


# speedup vs baseline: 1.3111x; 1.3111x over previous
import dataclasses
import functools

import jax
import jax.numpy as jnp
from jax import lax
from jax.experimental import pallas as pl
from jax.experimental.pallas import tpu as pltpu
from jax.experimental.pallas import tpu_sc as plsc

BATCH = 16384
EMBED_DIM = 64
PACKED = 2 * EMBED_DIM
N_USERS = 1000000
N_PRODUCTS = 100000

NC = 2
NS = 16
NW = NC * NS
BPW = BATCH // NW
CHUNK = 128
CPW = BPW // CHUNK

_sc_mesh = plsc.VectorSubcoreMesh(core_axis_name="c", subcore_axis_name="s")

_sc_params = pltpu.CompilerParams()
if "needs_layout_passes" in pltpu.CompilerParams.__dataclass_fields__:
    _sc_params = dataclasses.replace(_sc_params, needs_layout_passes=False)


def _repack_body(a, b, out):
    out[...] = jnp.concatenate([a[...], b[...]], axis=1)


def _tc_repack(table, blk):
    nmaj = table.shape[0] // 2
    nblk = nmaj // blk
    return pl.pallas_call(
        _repack_body,
        grid=(nblk,),
        in_specs=[pl.BlockSpec((blk, EMBED_DIM), lambda i: (i, 0)),
                  pl.BlockSpec((blk, EMBED_DIM),
                               lambda i, _n=nblk: (i + _n, 0))],
        out_specs=pl.BlockSpec((blk, PACKED), lambda i: (i, 0)),
        out_shape=jax.ShapeDtypeStruct((nmaj, PACKED), jnp.float32),
    )(table, table)


def _sc_gather(packed, qidx):
    @functools.partial(
        pl.kernel,
        mesh=_sc_mesh,
        compiler_params=_sc_params,
        out_type=jax.ShapeDtypeStruct((BATCH, PACKED), jnp.float32),
        scratch_types=[
            pltpu.VMEM((BPW,), jnp.int32),
            pltpu.VMEM((CHUNK, PACKED), jnp.float32),
            pltpu.SemaphoreType.DMA,
        ],
    )
    def k(t_hbm, q_hbm, o_hbm, q_v, r_v, sem):
        wid = lax.axis_index("s") * NC + lax.axis_index("c")
        base = wid * BPW
        pltpu.sync_copy(q_hbm.at[pl.ds(base, BPW)], q_v)
        for c in range(CPW):
            pltpu.async_copy(
                t_hbm.at[q_v.at[pl.ds(c * CHUNK, CHUNK)]], r_v, sem).wait()
            pltpu.sync_copy(r_v, o_hbm.at[pl.ds(base + c * CHUNK, CHUNK)])

    return k(packed, qidx)


def _mlp_body(bu, bp, glue, w1, b1, w2, b2, w3a, w3b, w3c, b3, w4, b4, out):
    g = glue[...]
    bu_ = bu[...]
    bp_ = bp[...]
    ue = jnp.where(g[:, 0:1] > 0, bu_[:, EMBED_DIM:], bu_[:, :EMBED_DIM])
    pe = jnp.where(g[:, 1:2] > 0, bp_[:, EMBED_DIM:], bp_[:, :EMBED_DIM])
    m = ue * pe
    uf = g[:, 2:13]
    bd = g[:, 13:16]
    ufeat = jnp.maximum(
        jnp.dot(uf, w1[...], preferred_element_type=jnp.float32) + b1[...], 0.0)
    bfeat = jnp.maximum(
        jnp.dot(bd, w2[...], preferred_element_type=jnp.float32) + b2[...], 0.0)
    h = (jnp.dot(m, w3a[...], preferred_element_type=jnp.float32)
         + jnp.dot(ufeat, w3b[...], preferred_element_type=jnp.float32)
         + jnp.dot(bfeat, w3c[...], preferred_element_type=jnp.float32)
         + b3[...])
    h = jnp.maximum(h, 0.0)
    logit = jnp.dot(h, w4[...], preferred_element_type=jnp.float32) + b4[...]
    out[...] = jax.nn.sigmoid(logit)


_TC_BLOCK = 2048


def _tc_mlp(bu, bp, glue, w1, b1, w2, b2, w3a, w3b, w3c, b3, w4, b4):
    def row_block(width):
        return pl.BlockSpec((_TC_BLOCK, width), lambda i: (i, 0))

    def whole(a):
        return pl.BlockSpec(a.shape, lambda i: (0, 0))

    return pl.pallas_call(
        _mlp_body,
        grid=(BATCH // _TC_BLOCK,),
        in_specs=[row_block(PACKED), row_block(PACKED), row_block(16),
                  whole(w1), whole(b1), whole(w2), whole(b2),
                  whole(w3a), whole(w3b), whole(w3c), whole(b3),
                  whole(w4), whole(b4)],
        out_specs=row_block(1),
        out_shape=jax.ShapeDtypeStruct((BATCH, 1), jnp.float32),
    )(bu, bp, glue, w1, b1, w2, b2, w3a, w3b, w3c, b3, w4, b4)


@jax.jit
def _run(user_ids, product_ids, user_features, behavior_data,
         user_table, product_table, W1, b1, W2, b2, W3, b3, W4, b4):
    hu = N_USERS // 2
    hp = N_PRODUCTS // 2
    uq = jnp.where(user_ids >= hu, user_ids - hu, user_ids)
    pq = jnp.where(product_ids >= hp, product_ids - hp, product_ids)
    su = (user_ids >= hu).astype(jnp.float32).reshape(BATCH, 1)
    sp = (product_ids >= hp).astype(jnp.float32).reshape(BATCH, 1)
    glue = jnp.concatenate([su, sp, user_features, behavior_data], axis=1)

    packed_p = _tc_repack(product_table, 5000)
    bp = _sc_gather(packed_p, pq)
    packed_u = _tc_repack(user_table, 5000)
    bu = _sc_gather(packed_u, uq)

    return _tc_mlp(
        bu, bp, glue,
        W1.T, b1.reshape(1, 32), W2.T, b2.reshape(1, 32),
        W3[:, :EMBED_DIM].T, W3[:, EMBED_DIM:EMBED_DIM + 32].T,
        W3[:, EMBED_DIM + 32:].T, b3.reshape(1, 32),
        W4.T, b4.reshape(1, 1))


def kernel(user_ids, product_ids, user_features, behavior_data,
           user_table, product_table, W1, b1, W2, b2, W3, b3, W4, b4):
    return _run(user_ids, product_ids, user_features, behavior_data,
                user_table, product_table, W1, b1, W2, b2, W3, b3, W4, b4)
